# Initial kernel scaffold; baseline (speedup 1.0000x reference)
import jax
import jax.numpy as jnp
from jax.experimental import pallas as pl


def kernel(x, embeddings):
    raise NotImplementedError("write your pallas kernel here")


# speedup vs baseline: 1.1097x; 1.1097x over previous
import functools

import jax
import jax.numpy as jnp
from jax import lax
from jax.experimental import pallas as pl
from jax.experimental.pallas import tpu as pltpu
from jax.experimental.pallas import tpu_sc as plsc

VOCAB = 1000000
DIM = 32
BATCH = 16384
HIST = 50
N = BATCH * HIST

_info = plsc.get_sparse_core_info()
NC, NS = _info.num_cores, _info.num_subcores
NW = NC * NS
BPW = N // NW
CHUNK = 3200
NCHUNK = BPW // CHUNK

_mesh = plsc.VectorSubcoreMesh(core_axis_name="c", subcore_axis_name="s")


@functools.partial(
    pl.kernel,
    out_type=jax.ShapeDtypeStruct((N, DIM), jnp.float32),
    mesh=_mesh,
    scratch_types=[
        pltpu.VMEM((CHUNK,), jnp.int32),
        pltpu.VMEM((CHUNK, DIM), jnp.float32),
        pltpu.SemaphoreType.DMA,
    ],
    compiler_params=pltpu.CompilerParams(use_tc_tiling_on_sc=False),
)
def _gather_rows(idx_hbm, tab_hbm, out_hbm, idx_v, rows_v, sem):
    wid = lax.axis_index("s") * NC + lax.axis_index("c")
    base = wid * BPW

    def chunk_body(i, carry):
        off = base + i * CHUNK
        pltpu.sync_copy(idx_hbm.at[pl.ds(off, CHUNK)], idx_v)
        pltpu.async_copy(tab_hbm.at[idx_v], rows_v, sem).wait()
        pltpu.sync_copy(rows_v, out_hbm.at[pl.ds(off, CHUNK)])
        return carry

    lax.fori_loop(0, NCHUNK, chunk_body, 0)


def kernel(x, embeddings):
    flat = x.reshape(N)
    out = _gather_rows(flat, embeddings)
    return out.reshape(BATCH, HIST, DIM)


# speedup vs baseline: 1.1130x; 1.0030x over previous
import functools

import jax
import jax.numpy as jnp
from jax import lax
from jax.experimental import pallas as pl
from jax.experimental.pallas import tpu as pltpu
from jax.experimental.pallas import tpu_sc as plsc

VOCAB = 1000000
DIM = 32
BATCH = 16384
HIST = 50
N = BATCH * HIST

_info = plsc.get_sparse_core_info()
NC, NS = _info.num_cores, _info.num_subcores
NW = NC * NS
BPW = N // NW
CHUNK = 800
NBUF = 4
NCHUNK = BPW // CHUNK

_mesh = plsc.VectorSubcoreMesh(core_axis_name="c", subcore_axis_name="s")


@functools.partial(
    pl.kernel,
    out_type=jax.ShapeDtypeStruct((N, DIM), jnp.float32),
    mesh=_mesh,
    scratch_types=[
        pltpu.VMEM((BPW,), jnp.int32),
        pltpu.VMEM((NBUF, CHUNK, DIM), jnp.float32),
        pltpu.SemaphoreType.DMA((NBUF,)),
        pltpu.SemaphoreType.DMA((NBUF,)),
    ],
    compiler_params=pltpu.CompilerParams(use_tc_tiling_on_sc=False),
)
def _gather_rows(idx_hbm, tab_hbm, out_hbm, idx_v, rows_v, sem_g, sem_o):
    wid = lax.axis_index("s") * NC + lax.axis_index("c")
    base = wid * BPW
    pltpu.sync_copy(idx_hbm.at[pl.ds(base, BPW)], idx_v)

    def gather(i, b):
        return pltpu.async_copy(
            tab_hbm.at[idx_v.at[pl.ds(i * CHUNK, CHUNK)]],
            rows_v.at[b],
            sem_g.at[b],
        )

    def scatter(i, b):
        return pltpu.async_copy(
            rows_v.at[b],
            out_hbm.at[pl.ds(base + i * CHUNK, CHUNK)],
            sem_o.at[b],
        )

    in_g = [None] * NBUF
    in_o = [None] * NBUF
    for b in range(min(NBUF, NCHUNK)):
        in_g[b] = gather(b, b)
    for i in range(NCHUNK):
        b = i % NBUF
        in_g[b].wait()
        in_o[b] = scatter(i, b)
        nxt = i + NBUF
        if nxt < NCHUNK:
            in_o[b].wait()
            in_g[b] = gather(nxt, b)
    for i in range(max(0, NCHUNK - NBUF), NCHUNK):
        in_o[i % NBUF].wait()


def kernel(x, embeddings):
    flat = x.reshape(N)
    out = _gather_rows(flat, embeddings)
    return out.reshape(BATCH, HIST, DIM)


# speedup vs baseline: 1.7863x; 1.6049x over previous
import functools

import jax
import jax.numpy as jnp
from jax import lax
from jax.experimental import pallas as pl
from jax.experimental.pallas import tpu as pltpu
from jax.experimental.pallas import tpu_sc as plsc

VOCAB = 1000000
DIM = 32
BATCH = 16384
HIST = 50
N = BATCH * HIST

ROW_W = 16
RPL = DIM // ROW_W
TAB_ROWS = VOCAB * RPL
OUT_ROWS = N * RPL

_info = plsc.get_sparse_core_info()
NC, NS = _info.num_cores, _info.num_subcores
NW = NC * NS
BPW = N // NW
CHUNK = 800
NBUF = 3
NCHUNK = BPW // CHUNK

_mesh = plsc.VectorSubcoreMesh(core_axis_name="c", subcore_axis_name="s")


@functools.partial(
    pl.kernel,
    out_type=jax.ShapeDtypeStruct((OUT_ROWS, ROW_W), jnp.float32),
    mesh=_mesh,
    scratch_types=[
        pltpu.VMEM((BPW,), jnp.int32),
        pltpu.VMEM((NBUF, RPL * CHUNK), jnp.int32),
        pltpu.VMEM((NBUF, RPL * CHUNK, ROW_W), jnp.float32),
        pltpu.SemaphoreType.DMA((NBUF,)),
        pltpu.SemaphoreType.DMA((NBUF,)),
    ],
    compiler_params=pltpu.CompilerParams(
        use_tc_tiling_on_sc=False, needs_layout_passes=False
    ),
)
def _gather_rows(idx_hbm, tab_hbm, out_hbm, idx_v, idx2_v, rows_v, sem_g, sem_o):
    wid = lax.axis_index("s") * NC + lax.axis_index("c")
    base = wid * BPW
    pltpu.sync_copy(idx_hbm.at[pl.ds(base, BPW)], idx_v)

    lanes = lax.iota(jnp.int32, 16)

    def build_idx2(i, b):
        slot = idx2_v.at[b]

        def body(j, carry):
            seg = idx_v[pl.ds(i * CHUNK + j * 16, 16)]
            two = seg * 2
            pos = (j * 16 + lanes) * 2
            plsc.store_scatter(slot, [pos], two)
            plsc.store_scatter(slot, [pos + 1], two + 1)
            return carry

        lax.fori_loop(0, CHUNK // 16, body, 0)

    def gather(i, b):
        build_idx2(i, b)
        return pltpu.async_copy(
            tab_hbm.at[idx2_v.at[b]],
            rows_v.at[b],
            sem_g.at[b],
        )

    def flush(i, b):
        return pltpu.async_copy(
            rows_v.at[b],
            out_hbm.at[pl.ds((base + i * CHUNK) * RPL, RPL * CHUNK)],
            sem_o.at[b],
        )

    in_g = [None] * NBUF
    in_o = [None] * NBUF
    for b in range(min(NBUF, NCHUNK)):
        in_g[b] = gather(b, b)
    for i in range(NCHUNK):
        b = i % NBUF
        in_g[b].wait()
        in_o[b] = flush(i, b)
        nxt = i + NBUF
        if nxt < NCHUNK:
            in_o[b].wait()
            in_g[b] = gather(nxt, b)
    for i in range(max(0, NCHUNK - NBUF), NCHUNK):
        in_o[i % NBUF].wait()


def kernel(x, embeddings):
    flat = x.reshape(N)
    tab = embeddings.reshape(TAB_ROWS, ROW_W)
    out = _gather_rows(flat, tab)
    return out.reshape(BATCH, HIST, DIM)
